# baseline (speedup 1.0000x reference)
import jax
import jax.numpy as jnp
from jax import lax
from jax.experimental import pallas as pl
from jax.experimental.pallas import tpu as pltpu
from jax.experimental.pallas import tpu_sc as plsc

NC = 2
NS = 16
NW = NC * NS
L = 16

N = 100000
K = 16
IN_DIM = 28
EMB_DIM = 128
W = K + 1

C = 128
CHUNKS_PER_W = 25
B_PER_W = C * CHUNKS_PER_W
LAST_BASE = N - C


def _body(x_hbm, table_hbm, out_hbm, x_v, idx_v, out_v, sem):
    wid = lax.axis_index("s") * NC + lax.axis_index("c")
    base = wid * B_PER_W
    lane = lax.iota(jnp.int32, L)

    def chunk_body(ci, _):
        row0 = jnp.minimum(base + ci * C, LAST_BASE)
        pltpu.sync_copy(x_hbm.at[pl.ds(row0, C)], x_v)

        def idx_grp(g, _):
            rows = lane + g * L
            vals = plsc.load_gather(x_v, [rows, jnp.full((L,), K, jnp.int32)])
            idx_v[pl.ds(g * L, L)] = vals.astype(jnp.int32)
            return 0

        lax.fori_loop(0, C // L, idx_grp, 0, unroll=True)
        pltpu.async_copy(table_hbm.at[idx_v], out_v, sem).wait()

        def struct_grp(g, _):
            rows = lane + g * L
            for c in range(K):
                cols = jnp.full((L,), c, jnp.int32)
                vals = plsc.load_gather(x_v, [rows, cols])
                plsc.store_scatter(out_v, [rows, cols], vals)
            return 0

        lax.fori_loop(0, C // L, struct_grp, 0)
        pltpu.sync_copy(out_v, out_hbm.at[pl.ds(row0, C)])
        return 0

    lax.fori_loop(0, CHUNKS_PER_W, chunk_body, 0)


@jax.jit
def _run(x, table):
    table128 = jnp.pad(table, ((0, 0), (K, 0)))
    mesh = plsc.VectorSubcoreMesh(
        core_axis_name="c", subcore_axis_name="s", num_cores=NC, num_subcores=NS
    )
    return pl.kernel(
        _body,
        out_type=jax.ShapeDtypeStruct((N, EMB_DIM), jnp.float32),
        mesh=mesh,
        compiler_params=pltpu.CompilerParams(needs_layout_passes=False),
        scratch_types=[
            pltpu.VMEM((C, W), jnp.float32),
            pltpu.VMEM((C,), jnp.int32),
            pltpu.VMEM((C, EMB_DIM), jnp.float32),
            pltpu.SemaphoreType.DMA,
        ],
    )(x, table128)


def kernel(x, table):
    return _run(x, table)


# speedup vs baseline: 1.9959x; 1.9959x over previous
import jax
import jax.numpy as jnp
from jax import lax
from jax.experimental import pallas as pl
from jax.experimental.pallas import tpu as pltpu
from jax.experimental.pallas import tpu_sc as plsc

NC = 2
NS = 16
NW = NC * NS
L = 16

N = 100000
K = 16
IN_DIM = 28
EMB_DIM = 128
W = K + 1

C = 128
NCH = 26
B_PER_W = C * NCH
LAST_BASE = N - C


def _body(x_hbm, table_hbm, out_hbm, table_s,
          x_v0, x_v1, idx_v0, idx_v1, out_v0, out_v1,
          sx0, sx1, sg0, sg1, sw0, sw1):
    s = lax.axis_index("s")
    c = lax.axis_index("c")
    base = (s * NC + c) * B_PER_W
    lane = lax.iota(jnp.int32, L)
    xv = [x_v0, x_v1]
    iv = [idx_v0, idx_v1]
    ov = [out_v0, out_v1]
    sx = [sx0, sx1]
    sg = [sg0, sg1]
    sw = [sw0, sw1]

    def row0(i):
        return jnp.minimum(base + i * C, LAST_BASE)

    def x_copy(i, b):
        return pltpu.make_async_copy(x_hbm.at[pl.ds(row0(i), C)], xv[b], sx[b])

    def g_copy(b):
        return pltpu.make_async_copy(table_s.at[iv[b]], ov[b], sg[b])

    def w_copy(i, b):
        return pltpu.make_async_copy(ov[b], out_hbm.at[pl.ds(row0(i), C)], sw[b])

    def extract(b):
        def idx_grp(g, _):
            rows = lane + g * L
            vals = plsc.load_gather(xv[b], [rows, jnp.full((L,), K, jnp.int32)])
            iv[b][pl.ds(g * L, L)] = vals.astype(jnp.int32)
            return 0

        lax.fori_loop(0, C // L, idx_grp, 0, unroll=True)

    def struct(b):
        def struct_grp(g, _):
            rows = lane + g * L
            for cc in range(K):
                cols = jnp.full((L,), cc, jnp.int32)
                vals = plsc.load_gather(xv[b], [rows, cols])
                plsc.store_scatter(ov[b], [rows, cols], vals)
            return 0

        lax.fori_loop(0, C // L, struct_grp, 0)

    x_copy(0, 0).start()
    x_copy(1, 1).start()

    @pl.when(s == 0)
    def _():
        pltpu.sync_copy(table_hbm, table_s)

    plsc.subcore_barrier()

    x_copy(0, 0).wait()
    extract(0)
    g_copy(0).start()
    x_copy(1, 1).wait()
    extract(1)
    g_copy(1).start()
    g_copy(0).wait()
    struct(0)
    w_copy(0, 0).start()
    x_copy(2, 0).start()

    def half_step(i, b):
        o = 1 - b
        x_copy(i + 1, o).wait()
        extract(o)
        w_copy(i - 1, o).wait()
        g_copy(o).start()
        g_copy(b).wait()
        struct(b)
        w_copy(i, b).start()
        x_copy(i + 2, b).start()

    def pair(p, _):
        half_step(2 * p + 1, 1)
        half_step(2 * p + 2, 0)
        return 0

    lax.fori_loop(0, (NCH - 2) // 2, pair, 0)

    g_copy(1).wait()
    struct(1)
    w_copy(NCH - 1, 1).start()
    w_copy(NCH - 2, 0).wait()
    w_copy(NCH - 1, 1).wait()
    x_copy(NCH, 0).wait()


@jax.jit
def _run(x, table):
    table128 = jnp.pad(table, ((0, 0), (K, 0)))
    mesh = plsc.VectorSubcoreMesh(
        core_axis_name="c", subcore_axis_name="s", num_cores=NC, num_subcores=NS
    )
    return pl.kernel(
        _body,
        out_type=jax.ShapeDtypeStruct((N, EMB_DIM), jnp.float32),
        mesh=mesh,
        compiler_params=pltpu.CompilerParams(needs_layout_passes=False),
        scratch_types=[
            pltpu.VMEM_SHARED((IN_DIM, EMB_DIM), jnp.float32),
            pltpu.VMEM((C, W), jnp.float32),
            pltpu.VMEM((C, W), jnp.float32),
            pltpu.VMEM((C,), jnp.int32),
            pltpu.VMEM((C,), jnp.int32),
            pltpu.VMEM((C, EMB_DIM), jnp.float32),
            pltpu.VMEM((C, EMB_DIM), jnp.float32),
            pltpu.SemaphoreType.DMA,
            pltpu.SemaphoreType.DMA,
            pltpu.SemaphoreType.DMA,
            pltpu.SemaphoreType.DMA,
            pltpu.SemaphoreType.DMA,
            pltpu.SemaphoreType.DMA,
        ],
    )(x, table128)


def kernel(x, table):
    return _run(x, table)


# speedup vs baseline: 2.8131x; 1.4094x over previous
import jax
import jax.numpy as jnp
from jax import lax
from jax.experimental import pallas as pl
from jax.experimental.pallas import tpu as pltpu
from jax.experimental.pallas import tpu_sc as plsc

NC = 2
NS = 16
NW = NC * NS
L = 16

N = 100000
K = 16
IN_DIM = 28
EMB_DIM = 128
W = K + 1

C = 112
NCH = 28
B_PER_W = C * NCH
LAST_BASE = N - C
R = 4


def _body(x_hbm, table_hbm, out_hbm, table_s,
          x_v0, x_v1, x_v2, x_v3, idx_v0, idx_v1, idx_v2, idx_v3,
          out_v0, out_v1, out_v2, out_v3,
          sx0, sx1, sx2, sx3, sg0, sg1, sg2, sg3, sw0, sw1, sw2, sw3):
    s = lax.axis_index("s")
    c = lax.axis_index("c")
    base = (s * NC + c) * B_PER_W
    lane = lax.iota(jnp.int32, L)
    xv = [x_v0, x_v1, x_v2, x_v3]
    iv = [idx_v0, idx_v1, idx_v2, idx_v3]
    ov = [out_v0, out_v1, out_v2, out_v3]
    sx = [sx0, sx1, sx2, sx3]
    sg = [sg0, sg1, sg2, sg3]
    sw = [sw0, sw1, sw2, sw3]

    def row0(i):
        return jnp.minimum(base + i * C, LAST_BASE)

    def x_copy(i, b):
        return pltpu.make_async_copy(x_hbm.at[pl.ds(row0(i), C)], xv[b], sx[b])

    def g_copy(b):
        return pltpu.make_async_copy(table_s.at[iv[b]], ov[b], sg[b])

    def w_copy(i, b):
        return pltpu.make_async_copy(ov[b], out_hbm.at[pl.ds(row0(i), C)], sw[b])

    def extract(b):
        def idx_grp(g, _):
            rows = lane + g * L
            vals = plsc.load_gather(xv[b], [rows, jnp.full((L,), K, jnp.int32)])
            iv[b][pl.ds(g * L, L)] = vals.astype(jnp.int32)
            return 0

        lax.fori_loop(0, C // L, idx_grp, 0, unroll=True)

    def struct(b):
        def struct_grp(g, _):
            rows = lane + g * L
            for cc in range(K):
                cols = jnp.full((L,), cc, jnp.int32)
                vals = plsc.load_gather(xv[b], [rows, cols])
                plsc.store_scatter(ov[b], [rows, cols], vals)
            return 0

        lax.fori_loop(0, C // L, struct_grp, 0)

    for i in range(R):
        x_copy(i, i).start()

    @pl.when(s == 0)
    def _():
        pltpu.sync_copy(table_hbm, table_s)

    plsc.subcore_barrier()
    x_copy(0, 0).wait()
    extract(0)
    g_copy(0).start()

    def body(i, j, with_wb_wait):
        b = j % R
        prev = (j - 1) % R
        x_copy(i, b).wait()
        extract(b)
        if with_wb_wait:
            w_copy(i - R, b).wait()
        g_copy(b).start()
        g_copy(prev).wait()
        struct(prev)
        w_copy(i - 1, prev).start()
        x_copy(i + (R - 1), prev).start()

    for i in range(1, R):
        body(i, i, with_wb_wait=False)

    def quad(q, _):
        for j in range(R):
            body(q * R + R + j, j, with_wb_wait=True)
        return 0

    lax.fori_loop(0, (NCH - R) // R, quad, 0)

    last = (NCH - 1) % R
    g_copy(last).wait()
    struct(last)
    w_copy(NCH - 1, last).start()
    for i in range(NCH - R, NCH):
        w_copy(i, i % R).wait()
    for i in range(NCH, NCH + R - 1):
        x_copy(i, i % R).wait()


@jax.jit
def _run(x, table):
    table128 = jnp.pad(table, ((0, 0), (K, 0)))
    mesh = plsc.VectorSubcoreMesh(
        core_axis_name="c", subcore_axis_name="s", num_cores=NC, num_subcores=NS
    )
    return pl.kernel(
        _body,
        out_type=jax.ShapeDtypeStruct((N, EMB_DIM), jnp.float32),
        mesh=mesh,
        compiler_params=pltpu.CompilerParams(needs_layout_passes=False),
        scratch_types=[
            pltpu.VMEM_SHARED((IN_DIM, EMB_DIM), jnp.float32),
        ]
        + [pltpu.VMEM((C, W), jnp.float32)] * 4
        + [pltpu.VMEM((C,), jnp.int32)] * 4
        + [pltpu.VMEM((C, EMB_DIM), jnp.float32)] * 4
        + [pltpu.SemaphoreType.DMA] * 12,
    )(x, table128)


def kernel(x, table):
    return _run(x, table)


# speedup vs baseline: 2.8135x; 1.0001x over previous
import jax
import jax.numpy as jnp
from jax import lax
from jax.experimental import pallas as pl
from jax.experimental.pallas import tpu as pltpu
from jax.experimental.pallas import tpu_sc as plsc

NC = 2
NS = 16
NW = NC * NS
L = 16

N = 100000
K = 16
IN_DIM = 28
EMB_DIM = 128
W = K + 1

C = 112
NCH = 28
B_PER_W = C * NCH
LAST_BASE = N - C
R = 4


def _body(x_hbm, table_hbm, out_hbm, table_s,
          x_v0, x_v1, x_v2, x_v3, idx_v0, idx_v1, idx_v2, idx_v3,
          out_v0, out_v1, out_v2, out_v3,
          sx0, sx1, sx2, sx3, sg0, sg1, sg2, sg3, sw0, sw1, sw2, sw3):
    s = lax.axis_index("s")
    c = lax.axis_index("c")
    base = (s * NC + c) * B_PER_W
    lane = lax.iota(jnp.int32, L)
    xv = [x_v0, x_v1, x_v2, x_v3]
    iv = [idx_v0, idx_v1, idx_v2, idx_v3]
    ov = [out_v0, out_v1, out_v2, out_v3]
    sx = [sx0, sx1, sx2, sx3]
    sg = [sg0, sg1, sg2, sg3]
    sw = [sw0, sw1, sw2, sw3]

    def row0(i):
        return jnp.minimum(base + i * C, LAST_BASE)

    def x_copy(i, b):
        return pltpu.make_async_copy(x_hbm.at[pl.ds(row0(i), C)], xv[b], sx[b])

    def g_copy(b):
        return pltpu.make_async_copy(table_s.at[iv[b]], ov[b], sg[b])

    def w_copy(i, b):
        return pltpu.make_async_copy(ov[b], out_hbm.at[pl.ds(row0(i), C)], sw[b])

    def extract(b):
        def idx_grp(g, _):
            rows = lane + g * L
            vals = plsc.load_gather(xv[b], [rows, jnp.full((L,), K, jnp.int32)])
            iv[b][pl.ds(g * L, L)] = vals.astype(jnp.int32)
            return 0

        lax.fori_loop(0, C // L, idx_grp, 0, unroll=True)

    def struct(b):
        def struct_grp(g, _):
            rows = lane + g * L
            for cc in range(K):
                cols = jnp.full((L,), cc, jnp.int32)
                vals = plsc.load_gather(xv[b], [rows, cols])
                plsc.store_scatter(ov[b], [rows, cols], vals)
            return 0

        lax.fori_loop(0, C // L, struct_grp, 0)

    for i in range(R):
        x_copy(i, i).start()

    @pl.when(s == 0)
    def _():
        pltpu.sync_copy(table_hbm, table_s)

    plsc.subcore_barrier()
    x_copy(0, 0).wait()
    extract(0)
    g_copy(0).start()
    x_copy(1, 1).wait()
    extract(1)
    g_copy(1).start()

    def body(i, j, with_wb_wait):
        b2 = (j + 2) % R
        b = j % R
        x_copy(i + 2, b2).wait()
        extract(b2)
        if with_wb_wait:
            w_copy(i - 2, b2).wait()
        g_copy(b2).start()
        g_copy(b).wait()
        struct(b)
        w_copy(i, b).start()
        x_copy(i + R, b).start()

    for i in range(2):
        body(i, i, with_wb_wait=False)

    def quad(q, _):
        for j in range(R):
            body(q * R + 2 + j, (2 + j) % R, with_wb_wait=True)
        return 0

    lax.fori_loop(0, (NCH - R) // R, quad, 0)

    for i in range(NCH - 2, NCH):
        b = i % R
        g_copy(b).wait()
        struct(b)
        w_copy(i, b).start()
    for i in range(NCH - R, NCH):
        w_copy(i, i % R).wait()
    for i in range(NCH, NCH + 2):
        x_copy(i, i % R).wait()


@jax.jit
def _run(x, table):
    table128 = jnp.pad(table, ((0, 0), (K, 0)))
    mesh = plsc.VectorSubcoreMesh(
        core_axis_name="c", subcore_axis_name="s", num_cores=NC, num_subcores=NS
    )
    return pl.kernel(
        _body,
        out_type=jax.ShapeDtypeStruct((N, EMB_DIM), jnp.float32),
        mesh=mesh,
        compiler_params=pltpu.CompilerParams(needs_layout_passes=False),
        scratch_types=[
            pltpu.VMEM_SHARED((IN_DIM, EMB_DIM), jnp.float32),
        ]
        + [pltpu.VMEM((C, W), jnp.float32)] * 4
        + [pltpu.VMEM((C,), jnp.int32)] * 4
        + [pltpu.VMEM((C, EMB_DIM), jnp.float32)] * 4
        + [pltpu.SemaphoreType.DMA] * 12,
    )(x, table128)


def kernel(x, table):
    return _run(x, table)


# speedup vs baseline: 3.7106x; 1.3189x over previous
import jax
import jax.numpy as jnp
from jax import lax
from jax.experimental import pallas as pl
from jax.experimental.pallas import tpu as pltpu
from jax.experimental.pallas import tpu_sc as plsc

NC = 2
NS = 16
NW = NC * NS
L = 16

N = 100000
K = 16
IN_DIM = 28
EMB_DIM = 128
W = K + 1

C = 112
NCH = 28
B_PER_W = C * NCH
LAST_BASE = N - C
R = 4


def _body(x_hbm, table_hbm, out_hbm, table_s,
          x_v0, x_v1, x_v2, x_v3, idx_v0, idx_v1, idx_v2, idx_v3,
          out_v0, out_v1, out_v2, out_v3,
          sx0, sx1, sx2, sx3, sg0, sg1, sg2, sg3, sw0, sw1, sw2, sw3):
    s = lax.axis_index("s")
    c = lax.axis_index("c")
    base = (s * NC + c) * B_PER_W
    lane = lax.iota(jnp.int32, L)
    xv = [x_v0, x_v1, x_v2, x_v3]
    iv = [idx_v0, idx_v1, idx_v2, idx_v3]
    ov = [out_v0, out_v1, out_v2, out_v3]
    sx = [sx0, sx1, sx2, sx3]
    sg = [sg0, sg1, sg2, sg3]
    sw = [sw0, sw1, sw2, sw3]

    def row0(i):
        return jnp.minimum(base + i * C, LAST_BASE)

    def x_copy(i, b):
        return pltpu.make_async_copy(x_hbm.at[pl.ds(row0(i), C)], xv[b], sx[b])

    def g_copy(b):
        return pltpu.make_async_copy(table_s.at[iv[b]], ov[b], sg[b])

    def w_copy(i, b):
        return pltpu.make_async_copy(ov[b], out_hbm.at[pl.ds(row0(i), C)], sw[b])

    def extract(b):
        def idx_grp(g, _):
            rows = lane + g * L
            vals = plsc.load_gather(xv[b], [rows, jnp.full((L,), K, jnp.int32)])
            iv[b][pl.ds(g * L, L)] = vals.astype(jnp.int32)
            return 0

        lax.fori_loop(0, C // L, idx_grp, 0, unroll=True)

    def struct(b):
        def struct_row(r, _):
            ov[b][r, pl.ds(0, K)] = xv[b][r, pl.ds(0, K)]
            return 0

        lax.fori_loop(0, C, struct_row, 0, unroll=8)

    for i in range(R):
        x_copy(i, i).start()

    @pl.when(s == 0)
    def _():
        pltpu.sync_copy(table_hbm, table_s)

    plsc.subcore_barrier()
    x_copy(0, 0).wait()
    extract(0)
    g_copy(0).start()
    x_copy(1, 1).wait()
    extract(1)
    g_copy(1).start()

    def body(i, j, with_wb_wait):
        b2 = (j + 2) % R
        b = j % R
        x_copy(i + 2, b2).wait()
        extract(b2)
        if with_wb_wait:
            w_copy(i - 2, b2).wait()
        g_copy(b2).start()
        g_copy(b).wait()
        struct(b)
        w_copy(i, b).start()
        x_copy(i + R, b).start()

    for i in range(2):
        body(i, i, with_wb_wait=False)

    def quad(q, _):
        for j in range(R):
            body(q * R + 2 + j, (2 + j) % R, with_wb_wait=True)
        return 0

    lax.fori_loop(0, (NCH - R) // R, quad, 0)

    for i in range(NCH - 2, NCH):
        b = i % R
        g_copy(b).wait()
        struct(b)
        w_copy(i, b).start()
    for i in range(NCH - R, NCH):
        w_copy(i, i % R).wait()
    for i in range(NCH, NCH + 2):
        x_copy(i, i % R).wait()


@jax.jit
def _run(x, table):
    table128 = jnp.pad(table, ((0, 0), (K, 0)))
    mesh = plsc.VectorSubcoreMesh(
        core_axis_name="c", subcore_axis_name="s", num_cores=NC, num_subcores=NS
    )
    return pl.kernel(
        _body,
        out_type=jax.ShapeDtypeStruct((N, EMB_DIM), jnp.float32),
        mesh=mesh,
        compiler_params=pltpu.CompilerParams(needs_layout_passes=False),
        scratch_types=[
            pltpu.VMEM_SHARED((IN_DIM, EMB_DIM), jnp.float32),
        ]
        + [pltpu.VMEM((C, W), jnp.float32)] * 4
        + [pltpu.VMEM((C,), jnp.int32)] * 4
        + [pltpu.VMEM((C, EMB_DIM), jnp.float32)] * 4
        + [pltpu.SemaphoreType.DMA] * 12,
    )(x, table128)


def kernel(x, table):
    return _run(x, table)
